# baseline (speedup 1.0000x reference)
import functools

import jax
import jax.numpy as jnp
from jax import lax
from jax.experimental import pallas as pl
from jax.experimental.pallas import tpu as pltpu
from jax.experimental.pallas import tpu_sc as plsc

D = 768
C = 32
NBUF = 4


def _sc_gather(tokens_flat, table):
    B = tokens_flat.shape[0]
    info = plsc.get_sparse_core_info()
    num_cores, num_subcores = info.num_cores, info.num_subcores
    nw = num_cores * num_subcores
    b_per_w = B // nw
    n = b_per_w // C
    assert n >= 8 and (n - 4) % NBUF == 0
    mesh = plsc.VectorSubcoreMesh(core_axis_name="c", subcore_axis_name="s")

    @functools.partial(
        pl.kernel,
        mesh=mesh,
        out_type=jax.ShapeDtypeStruct((B, D), jnp.float32),
        scratch_types=[
            pltpu.VMEM((b_per_w,), jnp.int32),
        ] + [pltpu.VMEM((C, D), jnp.float32) for _ in range(NBUF)]
          + [pltpu.SemaphoreType.DMA for _ in range(2 * NBUF)],
    )
    def k(tok_hbm, table_hbm, out_hbm, idx_v, *bufs):
        rows = bufs[:NBUF]
        gsem = bufs[NBUF:2 * NBUF]
        osem = bufs[2 * NBUF:]
        wid = lax.axis_index("s") * num_cores + lax.axis_index("c")
        base = wid * b_per_w

        pltpu.sync_copy(tok_hbm.at[pl.ds(base, b_per_w)], idx_v)

        H = C // 2

        def start_gather(i, b):
            for h in (0, 1):
                pltpu.async_copy(
                    table_hbm.at[idx_v.at[pl.ds(i * C + h * H, H)]],
                    rows[b].at[pl.ds(h * H, H)], gsem[b])

        def wait_gather(i, b):
            for h in (0, 1):
                pltpu.make_async_copy(
                    table_hbm.at[idx_v.at[pl.ds(i * C + h * H, H)]],
                    rows[b].at[pl.ds(h * H, H)], gsem[b]).wait()

        def start_out(i, b):
            for h in (0, 1):
                pltpu.async_copy(
                    rows[b].at[pl.ds(h * H, H)],
                    out_hbm.at[pl.ds(base + i * C + h * H, H)], osem[b])

        def wait_out(i, b):
            for h in (0, 1):
                pltpu.make_async_copy(
                    rows[b].at[pl.ds(h * H, H)],
                    out_hbm.at[pl.ds(base + i * C + h * H, H)], osem[b]).wait()

        start_gather(0, 0)
        start_gather(1, 1)
        wait_gather(0, 0)
        start_out(0, 0)
        start_gather(2, 2)
        wait_gather(1, 1)
        start_out(1, 1)
        start_gather(3, 3)

        def body(step, carry):
            i0 = 2 + step * NBUF
            for t in range(NBUF):
                i = i0 + t
                b = (2 + t) % NBUF
                wait_gather(i, b)
                start_out(i, b)
                wait_out(i - 2, (b + 2) % NBUF)
                start_gather(i + 2, (b + 2) % NBUF)
            return carry

        lax.fori_loop(0, (n - 4) // NBUF, body, 0)

        for i in (n - 2, n - 1):
            b = i % NBUF
            wait_gather(i, b)
            start_out(i, b)
        for i in (n - 4, n - 3, n - 2, n - 1):
            wait_out(i, i % NBUF)

    return k(tokens_flat, table)


def kernel(tokens, table):
    bsz, seq = tokens.shape
    out = _sc_gather(tokens.reshape(-1).astype(jnp.int32), table)
    return out.reshape(bsz, seq, D)
